# baseline (speedup 1.0000x reference)
import functools

import numpy as np

import jax
import jax.numpy as jnp
from jax import lax
from jax.experimental import pallas as pl
from jax.experimental.pallas import tpu as pltpu
from jax.experimental.pallas import tpu_sc as plsc

B = 16384
F = 32
RPS = 128 // F
NC = 2
NS = 16
NW = NC * NS
BPW = B // NW
CHUNK = 128
NCHUNK = BPW // CHUNK
LANES = 16
BLKS = CHUNK // LANES

_mesh = plsc.VectorSubcoreMesh(core_axis_name="c", subcore_axis_name="s")


@functools.partial(
    pl.kernel,
    mesh=_mesh,
    compiler_params=pltpu.CompilerParams(use_tc_tiling_on_sc=False),
    out_type=jax.ShapeDtypeStruct((B,), jnp.float32),
    scratch_types=[
        pltpu.VMEM((NCHUNK, CHUNK), jnp.int32),
        pltpu.VMEM((NCHUNK, CHUNK), jnp.int32),
        pltpu.VMEM((NCHUNK, CHUNK), jnp.int32),
        pltpu.VMEM((NCHUNK, CHUNK), jnp.int32),
        pltpu.VMEM((CHUNK, 128), jnp.float32),
        pltpu.VMEM((CHUNK, 128), jnp.float32),
        pltpu.VMEM((BPW,), jnp.float32),
        pltpu.SemaphoreType.DMA,
    ],
)
def _mf_kernel(user_hbm, item_hbm, uf_hbm, if_hbm, out_hbm,
               uidx, iidx, usid, isid, usup, isup, outv, sem):
    wid = lax.axis_index("s") * NC + lax.axis_index("c")
    base = wid * BPW

    for j in range(NCHUNK):
        pltpu.sync_copy(user_hbm.at[pl.ds(base + j * CHUNK, CHUNK)], uidx.at[j])
        pltpu.sync_copy(item_hbm.at[pl.ds(base + j * CHUNK, CHUNK)], iidx.at[j])
    for j in range(NCHUNK):
        for o in range(CHUNK // LANES):
            sl = pl.ds(o * LANES, LANES)
            usid[j, sl] = lax.shift_right_logical(uidx[j, sl], 2)
            isid[j, sl] = lax.shift_right_logical(iidx[j, sl], 2)

    lane = lax.iota(jnp.int32, LANES)

    def xlane(x, idx):
        return lax.gather(
            x, idx[:, None],
            lax.GatherDimensionNumbers(
                offset_dims=(), collapsed_slice_dims=(0,),
                start_index_map=(0,)),
            slice_sizes=(1,),
            mode=lax.GatherScatterMode.PROMISE_IN_BOUNDS)

    perms = [lane ^ d for d in (8, 4, 2, 1)]

    one_i = jnp.ones((LANES,), jnp.int32)

    def quarter_masks(qb):
        return [
            jnp.minimum(jnp.abs(qb - t), one_i).astype(jnp.float32)
            for t in range(RPS)
        ]

    def select_quarter(sup, row, masks, h):
        val = sup[row, pl.ds(h * LANES, LANES)]
        val = val - masks[0] * val
        for t in range(1, RPS):
            cand = sup[row, pl.ds(t * F + h * LANES, LANES)]
            val = val + (cand - masks[t] * cand)
        return val

    for j in range(NCHUNK):
        cp_u = pltpu.async_copy(uf_hbm.at[usid.at[j]], usup, sem)
        cp_v = pltpu.async_copy(if_hbm.at[isid.at[j]], isup, sem)
        cp_u.wait()
        cp_v.wait()

        def block(bb, carry):
            rbase = bb * LANES
            uq = uidx[j, pl.ds(rbase, LANES)] & (RPS - 1)
            vq = iidx[j, pl.ds(rbase, LANES)] & (RPS - 1)
            acc = jnp.zeros((LANES,), jnp.float32)
            for r in range(LANES):
                row = rbase + r
                rconst = jnp.full((LANES,), r, jnp.int32)
                um = quarter_masks(xlane(uq, rconst))
                vm = quarter_masks(xlane(vq, rconst))
                u0 = select_quarter(usup, row, um, 0)
                u1 = select_quarter(usup, row, um, 1)
                v0 = select_quarter(isup, row, vm, 0)
                v1 = select_quarter(isup, row, vm, 1)
                p = u0 * v0 + u1 * v1
                for pm in perms:
                    p = p + xlane(p, pm)
                oh = jnp.minimum(jnp.abs(lane - r), one_i).astype(jnp.float32)
                acc = acc + (p - oh * p)
            outv[pl.ds(j * CHUNK + rbase, LANES)] = acc
            return carry

        lax.fori_loop(0, BLKS, block, 0)

    pltpu.sync_copy(outv, out_hbm.at[pl.ds(base, BPW)])


def kernel(user, item, user_factors, item_factors):
    uf = user_factors.reshape(-1, 128)
    itf = item_factors.reshape(-1, 128)
    return _mf_kernel(user.astype(jnp.int32), item.astype(jnp.int32), uf, itf)


# speedup vs baseline: 1.0001x; 1.0001x over previous
import functools

import numpy as np

import jax
import jax.numpy as jnp
from jax import lax
from jax.experimental import pallas as pl
from jax.experimental.pallas import tpu as pltpu
from jax.experimental.pallas import tpu_sc as plsc

B = 16384
F = 32
RPS = 128 // F
NC = 2
NS = 16
NW = NC * NS
BPW = B // NW
CHUNK = 128
NCHUNK = BPW // CHUNK
LANES = 16
BLKS = CHUNK // LANES

_mesh = plsc.VectorSubcoreMesh(core_axis_name="c", subcore_axis_name="s")


@functools.partial(
    pl.kernel,
    mesh=_mesh,
    out_type=jax.ShapeDtypeStruct((B,), jnp.float32),
    scratch_types=[
        pltpu.VMEM((NCHUNK, CHUNK), jnp.int32),
        pltpu.VMEM((NCHUNK, CHUNK), jnp.int32),
        pltpu.VMEM((NCHUNK, CHUNK), jnp.int32),
        pltpu.VMEM((NCHUNK, CHUNK), jnp.int32),
        pltpu.VMEM((CHUNK, 128), jnp.float32),
        pltpu.VMEM((CHUNK, 128), jnp.float32),
        pltpu.VMEM((BPW,), jnp.float32),
        pltpu.SemaphoreType.DMA,
    ],
)
def _mf_kernel(user_hbm, item_hbm, uf_hbm, if_hbm, out_hbm,
               uidx, iidx, usid, isid, usup, isup, outv, sem):
    wid = lax.axis_index("s") * NC + lax.axis_index("c")
    base = wid * BPW

    for j in range(NCHUNK):
        pltpu.sync_copy(user_hbm.at[pl.ds(base + j * CHUNK, CHUNK)], uidx.at[j])
        pltpu.sync_copy(item_hbm.at[pl.ds(base + j * CHUNK, CHUNK)], iidx.at[j])
    for j in range(NCHUNK):
        for o in range(CHUNK // LANES):
            sl = pl.ds(o * LANES, LANES)
            usid[j, sl] = lax.shift_right_logical(uidx[j, sl], 2)
            isid[j, sl] = lax.shift_right_logical(iidx[j, sl], 2)

    lane = lax.iota(jnp.int32, LANES)

    def xlane(x, idx):
        return lax.gather(
            x, idx[:, None],
            lax.GatherDimensionNumbers(
                offset_dims=(), collapsed_slice_dims=(0,),
                start_index_map=(0,)),
            slice_sizes=(1,),
            mode=lax.GatherScatterMode.PROMISE_IN_BOUNDS)

    perms = [lane ^ d for d in (8, 4, 2, 1)]

    one_i = jnp.ones((LANES,), jnp.int32)

    def quarter_masks(qb):
        return [
            jnp.minimum(jnp.abs(qb - t), one_i).astype(jnp.float32)
            for t in range(RPS)
        ]

    def select_quarter(sup, row, masks, h):
        val = sup[row, pl.ds(h * LANES, LANES)]
        val = val - masks[0] * val
        for t in range(1, RPS):
            cand = sup[row, pl.ds(t * F + h * LANES, LANES)]
            val = val + (cand - masks[t] * cand)
        return val

    for j in range(NCHUNK):
        cp_u = pltpu.async_copy(uf_hbm.at[usid.at[j]], usup, sem)
        cp_v = pltpu.async_copy(if_hbm.at[isid.at[j]], isup, sem)
        cp_u.wait()
        cp_v.wait()

        def block(bb, carry):
            rbase = bb * LANES
            uq = uidx[j, pl.ds(rbase, LANES)] & (RPS - 1)
            vq = iidx[j, pl.ds(rbase, LANES)] & (RPS - 1)
            acc = jnp.zeros((LANES,), jnp.float32)
            for r in range(LANES):
                row = rbase + r
                rconst = jnp.full((LANES,), r, jnp.int32)
                um = quarter_masks(xlane(uq, rconst))
                vm = quarter_masks(xlane(vq, rconst))
                u0 = select_quarter(usup, row, um, 0)
                u1 = select_quarter(usup, row, um, 1)
                v0 = select_quarter(isup, row, vm, 0)
                v1 = select_quarter(isup, row, vm, 1)
                p = u0 * v0 + u1 * v1
                for pm in perms:
                    p = p + xlane(p, pm)
                oh = jnp.minimum(jnp.abs(lane - r), one_i).astype(jnp.float32)
                acc = acc + (p - oh * p)
            outv[pl.ds(j * CHUNK + rbase, LANES)] = acc
            return carry

        lax.fori_loop(0, BLKS, block, 0)

    pltpu.sync_copy(outv, out_hbm.at[pl.ds(base, BPW)])


def kernel(user, item, user_factors, item_factors):
    uf = user_factors.reshape(-1, 128)
    itf = item_factors.reshape(-1, 128)
    return _mf_kernel(user.astype(jnp.int32), item.astype(jnp.int32), uf, itf)


# speedup vs baseline: 1.0216x; 1.0215x over previous
import functools

import jax
import jax.numpy as jnp
from jax import lax
from jax.experimental import pallas as pl
from jax.experimental.pallas import tpu as pltpu
from jax.experimental.pallas import tpu_sc as plsc

B = 16384
F = 32
NC = 2
NS = 16
NW = NC * NS
BPW = B // NW
CHUNK = 128
NCHUNK = BPW // CHUNK
LANES = 16
NBLK = BPW // LANES

_mesh = plsc.VectorSubcoreMesh(core_axis_name="c", subcore_axis_name="s")


@functools.partial(
    pl.kernel,
    mesh=_mesh,
    compiler_params=pltpu.CompilerParams(use_tc_tiling_on_sc=False),
    out_type=jax.ShapeDtypeStruct((B,), jnp.float32),
    scratch_types=[
        pltpu.VMEM((NCHUNK, CHUNK), jnp.int32),
        pltpu.VMEM((NCHUNK, CHUNK), jnp.int32),
        pltpu.VMEM((BPW, F), jnp.float32),
        pltpu.VMEM((BPW, F), jnp.float32),
        pltpu.VMEM((BPW,), jnp.float32),
        pltpu.SemaphoreType.DMA,
    ],
)
def _mf_kernel(user_hbm, item_hbm, uf_hbm, if_hbm, out_hbm,
               uidx, iidx, urows, vrows, outv, sem):
    wid = lax.axis_index("s") * NC + lax.axis_index("c")
    base = wid * BPW

    for j in range(NCHUNK):
        pltpu.sync_copy(user_hbm.at[pl.ds(base + j * CHUNK, CHUNK)], uidx.at[j])
        pltpu.sync_copy(item_hbm.at[pl.ds(base + j * CHUNK, CHUNK)], iidx.at[j])

    copies = []
    for j in range(NCHUNK):
        copies.append(pltpu.async_copy(
            uf_hbm.at[uidx.at[j]], urows.at[pl.ds(j * CHUNK, CHUNK)], sem))
        copies.append(pltpu.async_copy(
            if_hbm.at[iidx.at[j]], vrows.at[pl.ds(j * CHUNK, CHUNK)], sem))
    for cp in copies:
        cp.wait()

    lane = lax.iota(jnp.int32, LANES)
    one_i = jnp.ones((LANES,), jnp.int32)

    def xlane(x, idx):
        return lax.gather(
            x, idx[:, None],
            lax.GatherDimensionNumbers(
                offset_dims=(), collapsed_slice_dims=(0,),
                start_index_map=(0,)),
            slice_sizes=(1,),
            mode=lax.GatherScatterMode.PROMISE_IN_BOUNDS)

    perms = [lane ^ d for d in (8, 4, 2, 1)]

    def block(bb, carry):
        rbase = bb * LANES
        acc = jnp.zeros((LANES,), jnp.float32)
        for r in range(LANES):
            row = rbase + r
            u0 = urows[row, pl.ds(0, LANES)]
            u1 = urows[row, pl.ds(LANES, LANES)]
            v0 = vrows[row, pl.ds(0, LANES)]
            v1 = vrows[row, pl.ds(LANES, LANES)]
            p = u0 * v0 + u1 * v1
            for pm in perms:
                p = p + xlane(p, pm)
            oh = jnp.minimum(jnp.abs(lane - r), one_i).astype(jnp.float32)
            acc = acc + (p - oh * p)
        outv[pl.ds(rbase, LANES)] = acc
        return carry

    lax.fori_loop(0, NBLK, block, 0)

    pltpu.sync_copy(outv, out_hbm.at[pl.ds(base, BPW)])


def kernel(user, item, user_factors, item_factors):
    return _mf_kernel(user.astype(jnp.int32), item.astype(jnp.int32),
                      user_factors, item_factors)
